# Initial kernel scaffold; baseline (speedup 1.0000x reference)
import jax
import jax.numpy as jnp
from jax.experimental import pallas as pl
from jax.experimental.pallas import tpu as pltpu

_M = 256
_N = 256
_DIM = 64
_B = 1024
_SIGMA = 128.0
_DECAY = 1000.0

_BBLK = 512
_KBLK = 4096
_KB = (_M * _N) // _KBLK

_OBBLK = 64


def _bmu_kernel(x_ref, w_ref, out_ref, val_ref, idx_ref):
    k = pl.program_id(1)

    @pl.when(k == 0)
    def _init():
        val_ref[...] = jnp.full_like(val_ref[...], jnp.inf)
        idx_ref[...] = jnp.zeros_like(idx_ref[...])

    x = x_ref[...]
    w = w_ref[...]
    x2 = jnp.sum(x * x, axis=1, keepdims=True)
    w2 = jnp.sum(w * w, axis=1)[None, :]
    mm = jax.lax.dot_general(
        x, w, (((1,), (1,)), ((), ())),
        preferred_element_type=jnp.float32,
        precision=jax.lax.Precision.HIGHEST,
    )
    s = (x2 - 2.0 * mm) + w2
    bm = jnp.min(s, axis=1, keepdims=True)
    iota = jax.lax.broadcasted_iota(jnp.int32, s.shape, 1)
    bidx = jnp.min(jnp.where(s == bm, iota, jnp.int32(2**30)),
                   axis=1, keepdims=True) + k * _KBLK
    upd = bm < val_ref[...]
    idx_ref[...] = jnp.where(upd, bidx, idx_ref[...])
    val_ref[...] = jnp.where(upd, bm, val_ref[...])

    @pl.when(k == _KB - 1)
    def _done():
        out_ref[...] = idx_ref[...]


def _neigh_kernel(scale_ref, bmu_ref, out_ref):
    neg_inv_s2 = scale_ref[0, 0]
    idx = bmu_ref[...]
    bi = (idx // _N).astype(jnp.float32)
    bj = (idx % _N).astype(jnp.float32)
    ii = jax.lax.broadcasted_iota(jnp.float32, (_OBBLK, _M), 1)
    jj = jax.lax.broadcasted_iota(jnp.float32, (_OBBLK, _N), 1)
    di = ii - bi
    dj = jj - bj
    fi = jnp.exp((di * di) * neg_inv_s2)
    fj = jnp.exp((dj * dj) * neg_inv_s2)
    out_ref[...] = fi[:, :, None] * fj[:, None, :]


def kernel(batch, weights, locations, it):
    del locations
    bmu_idx = pl.pallas_call(
        _bmu_kernel,
        grid=(_B // _BBLK, _KB),
        in_specs=[
            pl.BlockSpec((_BBLK, _DIM), lambda b, k: (b, 0)),
            pl.BlockSpec((_KBLK, _DIM), lambda b, k: (k, 0)),
        ],
        out_specs=pl.BlockSpec((_BBLK, 1), lambda b, k: (b, 0)),
        out_shape=jax.ShapeDtypeStruct((_B, 1), jnp.int32),
        scratch_shapes=[
            pltpu.VMEM((_BBLK, 1), jnp.float32),
            pltpu.VMEM((_BBLK, 1), jnp.int32),
        ],
        compiler_params=pltpu.CompilerParams(
            dimension_semantics=("parallel", "arbitrary"),
        ),
    )(batch, weights)

    lr = jnp.exp(-jnp.asarray(it, jnp.float32) / _DECAY)
    sigma = _SIGMA * lr
    neg_inv_s2 = (-1.0 / (sigma * sigma)).astype(jnp.float32).reshape(1, 1)

    out = pl.pallas_call(
        _neigh_kernel,
        grid=(_B // _OBBLK,),
        in_specs=[
            pl.BlockSpec(memory_space=pltpu.SMEM),
            pl.BlockSpec((_OBBLK, 1), lambda b: (b, 0)),
        ],
        out_specs=pl.BlockSpec((_OBBLK, _M, _N), lambda b: (b, 0, 0)),
        out_shape=jax.ShapeDtypeStruct((_B, _M, _N), jnp.float32),
        compiler_params=pltpu.CompilerParams(
            dimension_semantics=("parallel",),
        ),
    )(neg_inv_s2, bmu_idx)
    return out.reshape(_B, _M * _N)


# speedup vs baseline: 1.2058x; 1.2058x over previous
import jax
import jax.numpy as jnp
from jax.experimental import pallas as pl
from jax.experimental.pallas import tpu as pltpu

_M = 256
_N = 256
_DIM = 64
_B = 1024
_SIGMA = 128.0
_DECAY = 1000.0

_BBLK = 512
_KBLK = 4096
_KB = (_M * _N) // _KBLK

_OBBLK = 64
_BIG = 2**30


def _bmu_kernel(x_ref, w_ref, out_ref, val_ref, idx_ref):
    k = pl.program_id(1)

    @pl.when(k == 0)
    def _init():
        val_ref[...] = jnp.full_like(val_ref[...], jnp.inf)
        idx_ref[...] = jnp.zeros_like(idx_ref[...])

    x = x_ref[...]
    w = w_ref[...]
    x2 = jnp.sum(x * x, axis=1, keepdims=True)
    w2 = jnp.sum(w * w, axis=1)[None, :]
    mm = jax.lax.dot_general(
        x.astype(jnp.bfloat16), w.astype(jnp.bfloat16),
        (((1,), (1,)), ((), ())),
        preferred_element_type=jnp.float32,
    )
    s = (x2 - 2.0 * mm) + w2
    m = jnp.min(s, axis=1, keepdims=True)
    iota = jax.lax.broadcasted_iota(jnp.int32, s.shape, 1)
    bi = jnp.min(jnp.where(s == m, iota, _BIG), axis=1, keepdims=True) + k * _KBLK
    better = m < val_ref[...]
    idx_ref[...] = jnp.where(better, bi, idx_ref[...])
    val_ref[...] = jnp.where(better, m, val_ref[...])

    @pl.when(k == _KB - 1)
    def _done():
        out_ref[...] = idx_ref[...]


def _neigh_kernel(scale_ref, bmu_ref, out_ref):
    neg_inv_s2 = scale_ref[0, 0]
    idx = bmu_ref[...]
    bi = (idx // _N).astype(jnp.float32)
    bj = (idx % _N).astype(jnp.float32)
    ii = jax.lax.broadcasted_iota(jnp.int32, (_OBBLK, _M), 1).astype(jnp.float32)
    jj = jax.lax.broadcasted_iota(jnp.int32, (_OBBLK, _N), 1).astype(jnp.float32)
    di = ii - bi
    dj = jj - bj
    fi = jnp.exp((di * di) * neg_inv_s2)
    fj = jnp.exp((dj * dj) * neg_inv_s2)
    out_ref[...] = fi[:, :, None] * fj[:, None, :]


def kernel(batch, weights, locations, it):
    del locations
    bmu_idx = pl.pallas_call(
        _bmu_kernel,
        grid=(_B // _BBLK, _KB),
        in_specs=[
            pl.BlockSpec((_BBLK, _DIM), lambda b, k: (b, 0)),
            pl.BlockSpec((_KBLK, _DIM), lambda b, k: (k, 0)),
        ],
        out_specs=pl.BlockSpec((_BBLK, 1), lambda b, k: (b, 0)),
        out_shape=jax.ShapeDtypeStruct((_B, 1), jnp.int32),
        scratch_shapes=[
            pltpu.VMEM((_BBLK, 1), jnp.float32),
            pltpu.VMEM((_BBLK, 1), jnp.int32),
        ],
        compiler_params=pltpu.CompilerParams(
            dimension_semantics=("parallel", "arbitrary"),
        ),
    )(batch, weights)

    lr = jnp.exp(-jnp.asarray(it, jnp.float32) / _DECAY)
    sigma = _SIGMA * lr
    neg_inv_s2 = (-1.0 / (sigma * sigma)).astype(jnp.float32).reshape(1, 1)

    out = pl.pallas_call(
        _neigh_kernel,
        grid=(_B // _OBBLK,),
        in_specs=[
            pl.BlockSpec(memory_space=pltpu.SMEM),
            pl.BlockSpec((_OBBLK, 1), lambda b: (b, 0)),
        ],
        out_specs=pl.BlockSpec((_OBBLK, _M, _N), lambda b: (b, 0, 0)),
        out_shape=jax.ShapeDtypeStruct((_B, _M, _N), jnp.float32),
        compiler_params=pltpu.CompilerParams(
            dimension_semantics=("parallel",),
        ),
    )(neg_inv_s2, bmu_idx)
    return out.reshape(_B, _M * _N)
